# Initial kernel scaffold; baseline (speedup 1.0000x reference)
import functools

import jax
import jax.numpy as jnp
from jax import lax
from jax.experimental import pallas as pl
from jax.experimental.pallas import tpu as pltpu
from jax.experimental.pallas import tpu_sc as plsc

N_NODES = 10000
N_EDGES = 320000
IN_FEATS = 128
HIDDEN = 64

NC = 2
NS = 16
NW = NC * NS
CHUNK = 128
CHUNKS_PER_W = 80
E_PAD = NW * CHUNKS_PER_W * CHUNK
ACC_ROWS = 10016
ROWS_PER_TILE_Z = ACC_ROWS // NS
ROWS_PER_TILE_O = N_NODES // NS
DEG_W = 8


def _seg_kernel_body(with_deg, p_hbm, src_hbm, dst_hbm, zeros_hbm, ones_hbm,
                     *refs):
    if with_deg:
        (out_s, out_d, src_v, dst_v, rows_v, ones_v, acc_s, deg_s, sem) = refs
    else:
        (out_s, src_v, dst_v, rows_v, acc_s, sem) = refs
    c = lax.axis_index("c")
    s = lax.axis_index("s")
    wid = s * NC + c

    pltpu.sync_copy(zeros_hbm.at[pl.ds(0, ROWS_PER_TILE_Z)],
                    acc_s.at[pl.ds(s * ROWS_PER_TILE_Z, ROWS_PER_TILE_Z)])
    if with_deg:
        pltpu.sync_copy(zeros_hbm.at[pl.ds(0, ROWS_PER_TILE_Z), pl.ds(0, DEG_W)],
                        deg_s.at[pl.ds(s * ROWS_PER_TILE_Z, ROWS_PER_TILE_Z)])
        pltpu.sync_copy(ones_hbm, ones_v)
    pltpu.sync_copy(src_hbm.at[wid], src_v)
    pltpu.sync_copy(dst_hbm.at[wid], dst_v)
    plsc.subcore_barrier()

    def body(j, carry):
        pltpu.async_copy(p_hbm.at[src_v.at[j]], rows_v, sem).wait()
        pltpu.sync_copy(rows_v, acc_s.at[dst_v.at[j]], add=True)
        if with_deg:
            pltpu.sync_copy(ones_v, deg_s.at[dst_v.at[j]], add=True)
        return carry

    lax.fori_loop(0, CHUNKS_PER_W, body, 0)
    plsc.subcore_barrier()

    pltpu.sync_copy(acc_s.at[pl.ds(s * ROWS_PER_TILE_O, ROWS_PER_TILE_O)],
                    out_s.at[c, pl.ds(s * ROWS_PER_TILE_O, ROWS_PER_TILE_O)])
    if with_deg:
        pltpu.sync_copy(deg_s.at[pl.ds(s * ROWS_PER_TILE_O, ROWS_PER_TILE_O)],
                        out_d.at[c, pl.ds(s * ROWS_PER_TILE_O, ROWS_PER_TILE_O)])


def _make_seg_kernel(with_deg):
    mesh = plsc.VectorSubcoreMesh(core_axis_name="c", subcore_axis_name="s")
    out_type = [jax.ShapeDtypeStruct((NC, N_NODES, HIDDEN), jnp.float32)]
    scratch = [
        pltpu.VMEM((CHUNKS_PER_W, CHUNK), jnp.int32),
        pltpu.VMEM((CHUNKS_PER_W, CHUNK), jnp.int32),
        pltpu.VMEM((CHUNK, HIDDEN), jnp.float32),
    ]
    if with_deg:
        out_type.append(jax.ShapeDtypeStruct((NC, N_NODES, DEG_W), jnp.float32))
        scratch.append(pltpu.VMEM((CHUNK, DEG_W), jnp.float32))
    scratch.append(pltpu.VMEM_SHARED((ACC_ROWS, HIDDEN), jnp.float32))
    if with_deg:
        scratch.append(pltpu.VMEM_SHARED((ACC_ROWS, DEG_W), jnp.float32))
    scratch.append(pltpu.SemaphoreType.DMA)
    return pl.kernel(
        functools.partial(_seg_kernel_body, with_deg),
        out_type=out_type, mesh=mesh, scratch_types=scratch,
        name="sage_seg_deg" if with_deg else "sage_seg")


_seg_with_deg = _make_seg_kernel(True)
_seg_plain = _make_seg_kernel(False)


def _k1_body(f_ref, wn_ref, ws_ref, b_ref, p_ref, z_ref):
    f = f_ref[...]
    p_ref[...] = jnp.dot(f, wn_ref[...], preferred_element_type=jnp.float32)
    z_ref[...] = jnp.dot(f, ws_ref[...],
                         preferred_element_type=jnp.float32) + b_ref[...]


def _k2_body(z_ref, sp_ref, dp_ref, wn_ref, ws_ref, b_ref, p_ref, z2_ref):
    sp = sp_ref[...]
    deg = dp_ref[0, :, 0:1] + dp_ref[1, :, 0:1]
    deg = jnp.maximum(deg, 1.0)
    h = jax.nn.relu(z_ref[...] + (sp[0] + sp[1]) / deg)
    p_ref[...] = jnp.dot(h, wn_ref[...], preferred_element_type=jnp.float32)
    z2_ref[...] = jnp.dot(h, ws_ref[...],
                          preferred_element_type=jnp.float32) + b_ref[...]


def _k3_body(z_ref, sp_ref, dp_ref, wo_ref, bo_ref, out_ref):
    sp = sp_ref[...]
    deg = dp_ref[0, :, 0:1] + dp_ref[1, :, 0:1]
    deg = jnp.maximum(deg, 1.0)
    h = jax.nn.relu(z_ref[...] + (sp[0] + sp[1]) / deg)
    out_ref[...] = jnp.dot(h, wo_ref[...],
                           preferred_element_type=jnp.float32) + bo_ref[...]


def kernel(feats, edge_index, W_self1, W_neigh1, b1, W_self2, W_neigh2, b2,
           W_out, b_out):
    src = edge_index[0].astype(jnp.int32)
    dst = edge_index[1].astype(jnp.int32)
    pad = E_PAD - N_EDGES
    src_w = jnp.concatenate(
        [src, jnp.zeros((pad,), jnp.int32)]).reshape(NW, CHUNKS_PER_W, CHUNK)
    dst_w = jnp.concatenate(
        [dst, jnp.full((pad,), N_NODES, jnp.int32)]).reshape(
            NW, CHUNKS_PER_W, CHUNK)
    zeros = jnp.zeros((ROWS_PER_TILE_Z + 16, HIDDEN), jnp.float32)
    ones = jnp.ones((CHUNK, DEG_W), jnp.float32)

    nodes = feats.shape[0]
    p1, z1 = pl.pallas_call(
        _k1_body,
        out_shape=[jax.ShapeDtypeStruct((nodes, HIDDEN), jnp.float32)] * 2,
    )(feats, W_neigh1, W_self1, b1.reshape(1, HIDDEN))

    s1p, degp = _seg_with_deg(p1, src_w, dst_w, zeros, ones)

    p2, z2 = pl.pallas_call(
        _k2_body,
        out_shape=[jax.ShapeDtypeStruct((nodes, HIDDEN), jnp.float32)] * 2,
    )(z1, s1p, degp, W_neigh2, W_self2, b2.reshape(1, HIDDEN))

    (s2p,) = _seg_plain(p2, src_w, dst_w, zeros)

    logits = pl.pallas_call(
        _k3_body,
        out_shape=jax.ShapeDtypeStruct((nodes, W_out.shape[1]), jnp.float32),
    )(z2, s2p, degp, W_out, b_out.reshape(1, W_out.shape[1]))
    return logits


# speedup vs baseline: 5.5057x; 5.5057x over previous
import functools

import jax
import jax.numpy as jnp
from jax import lax
from jax.experimental import pallas as pl
from jax.experimental.pallas import tpu as pltpu
from jax.experimental.pallas import tpu_sc as plsc

N_NODES = 10000
N_EDGES = 320000
IN_FEATS = 128
HIDDEN = 64

NC = 2
NS = 16
NW = NC * NS
CHUNK = 128
CHUNKS_PER_W = 80
E_PAD = NW * CHUNKS_PER_W * CHUNK
ACC_ROWS = 10112
RPT = ACC_ROWS // NS
DEG_W = 8


def _seg_kernel_body(with_deg, p_hbm, src_hbm, dst_hbm, zeros_hbm, *refs):
    if with_deg:
        (zeros_deg_hbm, ones_hbm, out_s, out_d, src_v, dst_v, rows_v, ones_v,
         acc_s, deg_s, sem) = refs
    else:
        (out_s, src_v, dst_v, rows_v, acc_s, sem) = refs
    c = lax.axis_index("c")
    s = lax.axis_index("s")
    wid = s * NC + c

    pltpu.sync_copy(zeros_hbm, acc_s.at[pl.ds(s * RPT, RPT)])
    if with_deg:
        pltpu.sync_copy(zeros_deg_hbm, deg_s.at[pl.ds(s * RPT, RPT)])
        pltpu.sync_copy(ones_hbm, ones_v)
    pltpu.sync_copy(src_hbm.at[wid], src_v)
    pltpu.sync_copy(dst_hbm.at[wid], dst_v)
    plsc.subcore_barrier()

    def body(j, carry):
        pltpu.async_copy(p_hbm.at[src_v.at[j]], rows_v, sem).wait()
        pltpu.sync_copy(rows_v, acc_s.at[dst_v.at[j]], add=True)
        if with_deg:
            pltpu.sync_copy(ones_v, deg_s.at[dst_v.at[j]], add=True)
        return carry

    lax.fori_loop(0, CHUNKS_PER_W, body, 0)
    plsc.subcore_barrier()

    pltpu.sync_copy(acc_s.at[pl.ds(s * RPT, RPT)],
                    out_s.at[c, pl.ds(s * RPT, RPT)])
    if with_deg:
        pltpu.sync_copy(deg_s.at[pl.ds(s * RPT, RPT)],
                        out_d.at[c, pl.ds(s * RPT, RPT)])


def _make_seg_kernel(with_deg):
    mesh = plsc.VectorSubcoreMesh(core_axis_name="c", subcore_axis_name="s")
    out_type = [jax.ShapeDtypeStruct((NC, ACC_ROWS, HIDDEN), jnp.float32)]
    scratch = [
        pltpu.VMEM((CHUNKS_PER_W, CHUNK), jnp.int32),
        pltpu.VMEM((CHUNKS_PER_W, CHUNK), jnp.int32),
        pltpu.VMEM((CHUNK, HIDDEN), jnp.float32),
    ]
    if with_deg:
        out_type.append(jax.ShapeDtypeStruct((NC, ACC_ROWS, DEG_W), jnp.float32))
        scratch.append(pltpu.VMEM((CHUNK, DEG_W), jnp.float32))
    scratch.append(pltpu.VMEM_SHARED((ACC_ROWS, HIDDEN), jnp.float32))
    if with_deg:
        scratch.append(pltpu.VMEM_SHARED((ACC_ROWS, DEG_W), jnp.float32))
    scratch.append(pltpu.SemaphoreType.DMA)
    return pl.kernel(
        functools.partial(_seg_kernel_body, with_deg),
        out_type=out_type, mesh=mesh, scratch_types=scratch,
        compiler_params=pltpu.CompilerParams(use_tc_tiling_on_sc=False),
        name="sage_seg_deg" if with_deg else "sage_seg")


_seg_with_deg = _make_seg_kernel(True)
_seg_plain = _make_seg_kernel(False)


def _k1_body(f_ref, wn_ref, ws_ref, b_ref, p_ref, z_ref):
    f = f_ref[...]
    p_ref[...] = jnp.dot(f, wn_ref[...], preferred_element_type=jnp.float32)
    z_ref[...] = jnp.dot(f, ws_ref[...],
                         preferred_element_type=jnp.float32) + b_ref[...]


def _k2_body(z_ref, sp_ref, dp_ref, wn_ref, ws_ref, b_ref, p_ref, z2_ref):
    sp = sp_ref[...]
    deg = dp_ref[0, :, 0:1] + dp_ref[1, :, 0:1]
    deg = jnp.maximum(deg, 1.0)
    h = jax.nn.relu(z_ref[...] + (sp[0] + sp[1]) / deg)
    p_ref[...] = jnp.dot(h, wn_ref[...], preferred_element_type=jnp.float32)
    z2_ref[...] = jnp.dot(h, ws_ref[...],
                          preferred_element_type=jnp.float32) + b_ref[...]


def _k3_body(z_ref, sp_ref, dp_ref, wo_ref, bo_ref, out_ref):
    sp = sp_ref[...]
    deg = dp_ref[0, :, 0:1] + dp_ref[1, :, 0:1]
    deg = jnp.maximum(deg, 1.0)
    h = jax.nn.relu(z_ref[...] + (sp[0] + sp[1]) / deg)
    out_ref[...] = jnp.dot(h, wo_ref[...],
                           preferred_element_type=jnp.float32) + bo_ref[...]


def kernel(feats, edge_index, W_self1, W_neigh1, b1, W_self2, W_neigh2, b2,
           W_out, b_out):
    src = edge_index[0].astype(jnp.int32)
    dst = edge_index[1].astype(jnp.int32)
    pad = E_PAD - N_EDGES
    src_w = jnp.concatenate(
        [src, jnp.zeros((pad,), jnp.int32)]).reshape(NW, CHUNKS_PER_W, CHUNK)
    dst_w = jnp.concatenate(
        [dst, jnp.full((pad,), N_NODES, jnp.int32)]).reshape(
            NW, CHUNKS_PER_W, CHUNK)
    zeros = jnp.zeros((RPT, HIDDEN), jnp.float32)
    zeros_deg = jnp.zeros((RPT, DEG_W), jnp.float32)
    ones = jnp.ones((CHUNK, DEG_W), jnp.float32)

    nodes = feats.shape[0]
    p1, z1 = pl.pallas_call(
        _k1_body,
        out_shape=[jax.ShapeDtypeStruct((nodes, HIDDEN), jnp.float32)] * 2,
    )(feats, W_neigh1, W_self1, b1.reshape(1, HIDDEN))

    s1p, degp = _seg_with_deg(p1, src_w, dst_w, zeros, zeros_deg, ones)
    s1p = s1p[:, :nodes]
    degp = degp[:, :nodes]

    p2, z2 = pl.pallas_call(
        _k2_body,
        out_shape=[jax.ShapeDtypeStruct((nodes, HIDDEN), jnp.float32)] * 2,
    )(z1, s1p, degp, W_neigh2, W_self2, b2.reshape(1, HIDDEN))

    (s2p,) = _seg_plain(p2, src_w, dst_w, zeros)
    s2p = s2p[:, :nodes]

    logits = pl.pallas_call(
        _k3_body,
        out_shape=jax.ShapeDtypeStruct((nodes, W_out.shape[1]), jnp.float32),
    )(z2, s2p, degp, W_out, b_out.reshape(1, W_out.shape[1]))
    return logits


# speedup vs baseline: 6.5261x; 1.1853x over previous
import functools

import jax
import jax.numpy as jnp
from jax import lax
from jax.experimental import pallas as pl
from jax.experimental.pallas import tpu as pltpu
from jax.experimental.pallas import tpu_sc as plsc

N_NODES = 10000
N_EDGES = 320000
IN_FEATS = 128
HIDDEN = 64

NC = 2
NS = 16
NW = NC * NS
CHUNK = 128
CHUNKS_PER_W = 80
E_PAD = NW * CHUNKS_PER_W * CHUNK
ACC_ROWS = 10112
RPT = ACC_ROWS // NS
DEG_W = 8


def _seg_kernel_body(with_deg, p_hbm, src_hbm, dst_hbm, zeros_hbm, *refs):
    if with_deg:
        (zeros_deg_hbm, ones_hbm, out_s, out_d, src_v, dst_v, rows0_v, rows1_v,
         ones_v, acc_s, deg_s, gsem0, gsem1, osem) = refs
    else:
        (out_s, src_v, dst_v, rows0_v, rows1_v, acc_s, gsem0, gsem1,
         osem) = refs
    c = lax.axis_index("c")
    s = lax.axis_index("s")
    wid = s * NC + c

    pltpu.sync_copy(zeros_hbm, acc_s.at[pl.ds(s * RPT, RPT)])
    if with_deg:
        pltpu.sync_copy(zeros_deg_hbm, deg_s.at[pl.ds(s * RPT, RPT)])
        pltpu.sync_copy(ones_hbm, ones_v)
    pltpu.sync_copy(src_hbm.at[wid], src_v)
    pltpu.sync_copy(dst_hbm.at[wid], dst_v)
    plsc.subcore_barrier()

    def gather(j, buf, sem):
        return pltpu.async_copy(p_hbm.at[src_v.at[j]], buf, sem)

    def drain(j, buf, sem):
        pltpu.make_async_copy(p_hbm.at[src_v.at[j]], buf, sem).wait()
        if with_deg:
            oc = pltpu.async_copy(ones_v, deg_s.at[dst_v.at[j]], osem,
                                  add=True)
            pltpu.sync_copy(buf, acc_s.at[dst_v.at[j]], add=True)
            oc.wait()
        else:
            pltpu.sync_copy(buf, acc_s.at[dst_v.at[j]], add=True)

    gather(0, rows0_v, gsem0)
    gather(1, rows1_v, gsem1)

    def body(g, carry):
        j = 2 * g
        drain(j, rows0_v, gsem0)
        gather(j + 2, rows0_v, gsem0)
        drain(j + 1, rows1_v, gsem1)
        gather(j + 3, rows1_v, gsem1)
        return carry

    lax.fori_loop(0, CHUNKS_PER_W // 2 - 1, body, 0)
    drain(CHUNKS_PER_W - 2, rows0_v, gsem0)
    drain(CHUNKS_PER_W - 1, rows1_v, gsem1)
    plsc.subcore_barrier()

    pltpu.sync_copy(acc_s.at[pl.ds(s * RPT, RPT)],
                    out_s.at[c, pl.ds(s * RPT, RPT)])
    if with_deg:
        pltpu.sync_copy(deg_s.at[pl.ds(s * RPT, RPT)],
                        out_d.at[c, pl.ds(s * RPT, RPT)])


def _make_seg_kernel(with_deg):
    mesh = plsc.VectorSubcoreMesh(core_axis_name="c", subcore_axis_name="s")
    out_type = [jax.ShapeDtypeStruct((NC, ACC_ROWS, HIDDEN), jnp.float32)]
    scratch = [
        pltpu.VMEM((CHUNKS_PER_W, CHUNK), jnp.int32),
        pltpu.VMEM((CHUNKS_PER_W, CHUNK), jnp.int32),
        pltpu.VMEM((CHUNK, HIDDEN), jnp.float32),
        pltpu.VMEM((CHUNK, HIDDEN), jnp.float32),
    ]
    if with_deg:
        out_type.append(jax.ShapeDtypeStruct((NC, ACC_ROWS, DEG_W), jnp.float32))
        scratch.append(pltpu.VMEM((CHUNK, DEG_W), jnp.float32))
    scratch.append(pltpu.VMEM_SHARED((ACC_ROWS, HIDDEN), jnp.float32))
    if with_deg:
        scratch.append(pltpu.VMEM_SHARED((ACC_ROWS, DEG_W), jnp.float32))
    scratch += [pltpu.SemaphoreType.DMA] * 3
    return pl.kernel(
        functools.partial(_seg_kernel_body, with_deg),
        out_type=out_type, mesh=mesh, scratch_types=scratch,
        compiler_params=pltpu.CompilerParams(use_tc_tiling_on_sc=False),
        name="sage_seg_deg" if with_deg else "sage_seg")


_seg_with_deg = _make_seg_kernel(True)
_seg_plain = _make_seg_kernel(False)


def _k1_body(f_ref, wn_ref, ws_ref, b_ref, p_ref, z_ref):
    f = f_ref[...]
    p_ref[...] = jnp.dot(f, wn_ref[...], preferred_element_type=jnp.float32)
    z_ref[...] = jnp.dot(f, ws_ref[...],
                         preferred_element_type=jnp.float32) + b_ref[...]


def _k2_body(z_ref, sp_ref, dp_ref, wn_ref, ws_ref, b_ref, p_ref, z2_ref):
    sp = sp_ref[...]
    deg = dp_ref[0, :, 0:1] + dp_ref[1, :, 0:1]
    deg = jnp.maximum(deg, 1.0)
    h = jax.nn.relu(z_ref[...] + (sp[0] + sp[1]) / deg)
    p_ref[...] = jnp.dot(h, wn_ref[...], preferred_element_type=jnp.float32)
    z2_ref[...] = jnp.dot(h, ws_ref[...],
                          preferred_element_type=jnp.float32) + b_ref[...]


def _k3_body(z_ref, sp_ref, dp_ref, wo_ref, bo_ref, out_ref):
    sp = sp_ref[...]
    deg = dp_ref[0, :, 0:1] + dp_ref[1, :, 0:1]
    deg = jnp.maximum(deg, 1.0)
    h = jax.nn.relu(z_ref[...] + (sp[0] + sp[1]) / deg)
    out_ref[...] = jnp.dot(h, wo_ref[...],
                           preferred_element_type=jnp.float32) + bo_ref[...]


def kernel(feats, edge_index, W_self1, W_neigh1, b1, W_self2, W_neigh2, b2,
           W_out, b_out):
    src = edge_index[0].astype(jnp.int32)
    dst = edge_index[1].astype(jnp.int32)
    pad = E_PAD - N_EDGES
    src_w = jnp.concatenate(
        [src, jnp.zeros((pad,), jnp.int32)]).reshape(NW, CHUNKS_PER_W, CHUNK)
    dst_w = jnp.concatenate(
        [dst, jnp.full((pad,), N_NODES, jnp.int32)]).reshape(
            NW, CHUNKS_PER_W, CHUNK)
    zeros = jnp.zeros((RPT, HIDDEN), jnp.float32)
    zeros_deg = jnp.zeros((RPT, DEG_W), jnp.float32)
    ones = jnp.ones((CHUNK, DEG_W), jnp.float32)

    nodes = feats.shape[0]
    p1, z1 = pl.pallas_call(
        _k1_body,
        out_shape=[jax.ShapeDtypeStruct((nodes, HIDDEN), jnp.float32)] * 2,
    )(feats, W_neigh1, W_self1, b1.reshape(1, HIDDEN))

    s1p, degp = _seg_with_deg(p1, src_w, dst_w, zeros, zeros_deg, ones)
    s1p = s1p[:, :nodes]
    degp = degp[:, :nodes]

    p2, z2 = pl.pallas_call(
        _k2_body,
        out_shape=[jax.ShapeDtypeStruct((nodes, HIDDEN), jnp.float32)] * 2,
    )(z1, s1p, degp, W_neigh2, W_self2, b2.reshape(1, HIDDEN))

    (s2p,) = _seg_plain(p2, src_w, dst_w, zeros)
    s2p = s2p[:, :nodes]

    logits = pl.pallas_call(
        _k3_body,
        out_shape=jax.ShapeDtypeStruct((nodes, W_out.shape[1]), jnp.float32),
    )(z2, s2p, degp, W_out, b_out.reshape(1, W_out.shape[1]))
    return logits
